# initial kernel scaffold (unmeasured)
import jax
import jax.numpy as jnp
from jax import lax
from jax.experimental import pallas as pl
from jax.experimental.pallas import tpu as pltpu

N_DEV = 4


def kernel(x, w_mat):
    m, k_shard = x.shape
    _, n = w_mat.shape
    chunk = m // N_DEV

    def body(x_hbm, w_hbm, out_hbm, w_vmem, x_vmem, send_buf, recv_buf,
             local_sem, rs_send_sems, rs_recv_sems, ag_send_sems,
             ag_recv_sems, credit_sem):
        r = lax.axis_index("i")
        right = lax.rem(r + 1, N_DEV)
        left = lax.rem(r + N_DEV - 1, N_DEV)

        bsem = pltpu.get_barrier_semaphore()
        pl.semaphore_signal(bsem, inc=1, device_id=(left,),
                            device_id_type=pl.DeviceIdType.MESH)
        pl.semaphore_signal(bsem, inc=1, device_id=(right,),
                            device_id_type=pl.DeviceIdType.MESH)
        pl.semaphore_wait(bsem, 2)

        cw = pltpu.make_async_copy(w_hbm, w_vmem, local_sem)
        cw.start()
        cw.wait()

        def load_x_chunk(c):
            cp = pltpu.make_async_copy(
                x_hbm.at[pl.ds(c * chunk, chunk), :], x_vmem, local_sem)
            cp.start()
            cp.wait()

        def partial(c):
            load_x_chunk(c)
            return jnp.dot(x_vmem[...], w_vmem[...],
                           preferred_element_type=jnp.float32)

        send_buf[...] = partial(lax.rem(r + N_DEV - 1, N_DEV))
        for s in range(N_DEV - 1):
            if s > 0:
                pl.semaphore_wait(credit_sem, 1)
            rdma = pltpu.make_async_remote_copy(
                src_ref=send_buf, dst_ref=recv_buf,
                send_sem=rs_send_sems.at[s], recv_sem=rs_recv_sems.at[s],
                device_id=(right,), device_id_type=pl.DeviceIdType.MESH)
            rdma.start()
            rdma.wait()
            c = lax.rem(r + ((-2 - s) % N_DEV), N_DEV)
            send_buf[...] = partial(c) + recv_buf[...]
            if s < N_DEV - 2:
                pl.semaphore_signal(credit_sem, inc=1, device_id=(left,),
                                    device_id_type=pl.DeviceIdType.MESH)

        co = pltpu.make_async_copy(
            send_buf, out_hbm.at[pl.ds(r * chunk, chunk), :], local_sem)
        co.start()
        co.wait()

        for t in range(N_DEV - 1):
            a = lax.rem(r + ((-t) % N_DEV), N_DEV)
            rdma = pltpu.make_async_remote_copy(
                src_ref=out_hbm.at[pl.ds(a * chunk, chunk), :],
                dst_ref=out_hbm.at[pl.ds(a * chunk, chunk), :],
                send_sem=ag_send_sems.at[t], recv_sem=ag_recv_sems.at[t],
                device_id=(right,), device_id_type=pl.DeviceIdType.MESH)
            rdma.start()
            rdma.wait()

    return pl.pallas_call(
        body,
        out_shape=jax.ShapeDtypeStruct((m, n), jnp.float32),
        in_specs=[pl.BlockSpec(memory_space=pl.ANY),
                  pl.BlockSpec(memory_space=pl.ANY)],
        out_specs=pl.BlockSpec(memory_space=pl.ANY),
        scratch_shapes=[
            pltpu.VMEM((k_shard, n), jnp.bfloat16),
            pltpu.VMEM((chunk, k_shard), jnp.bfloat16),
            pltpu.VMEM((chunk, n), jnp.float32),
            pltpu.VMEM((chunk, n), jnp.float32),
            pltpu.SemaphoreType.DMA,
            pltpu.SemaphoreType.DMA((N_DEV - 1,)),
            pltpu.SemaphoreType.DMA((N_DEV - 1,)),
            pltpu.SemaphoreType.DMA((N_DEV - 1,)),
            pltpu.SemaphoreType.DMA((N_DEV - 1,)),
            pltpu.SemaphoreType.REGULAR,
        ],
        compiler_params=pltpu.CompilerParams(collective_id=0),
    )(x, w_mat)


# baseline (device time: 2504220 ns/iter reference)
import jax
import jax.numpy as jnp
from jax import lax
from jax.experimental import pallas as pl
from jax.experimental.pallas import tpu as pltpu

jax.config.update("jax_compilation_cache_dir", "/tmp/jaxcache")
jax.config.update("jax_persistent_cache_min_compile_time_secs", 1.0)

N_DEV = 4
NB = 2


def kernel(x, w_mat):
    x = x.astype(jnp.bfloat16)
    w_mat = w_mat.astype(jnp.bfloat16)
    m, k_shard = x.shape
    _, n = w_mat.shape
    chunk = m // N_DEV
    band = chunk // NB

    def body(x_hbm, w_hbm, out_hbm, w_vmem, x_vmem, send_buf, recv_buf,
             local_sem, rs_send_sems, rs_recv_sems, ag_send_sems,
             ag_recv_sems, credit_sem):
        j = pl.program_id(0)
        r = lax.axis_index("i")
        right = lax.rem(r + 1, N_DEV)
        left = lax.rem(r + N_DEV - 1, N_DEV)

        @pl.when(j == 0)
        def _():
            bsem = pltpu.get_barrier_semaphore()
            pl.semaphore_signal(bsem, inc=1, device_id=(left,),
                                device_id_type=pl.DeviceIdType.MESH)
            pl.semaphore_signal(bsem, inc=1, device_id=(right,),
                                device_id_type=pl.DeviceIdType.MESH)
            pl.semaphore_wait(bsem, 2)
            cw = pltpu.make_async_copy(w_hbm, w_vmem, local_sem)
            cw.start()
            cw.wait()

        def partial(c):
            cp = pltpu.make_async_copy(
                x_hbm.at[pl.ds(c * chunk + j * band, band), :], x_vmem,
                local_sem)
            cp.start()
            cp.wait()
            return jnp.dot(x_vmem[...], w_vmem[...],
                           preferred_element_type=jnp.float32)

        send_buf[...] = partial(lax.rem(r + N_DEV - 1, N_DEV))

        def rs_step(s, carry):
            @pl.when(jnp.logical_or(j > 0, s > 0))
            def _():
                pl.semaphore_wait(credit_sem, 1)

            rdma = pltpu.make_async_remote_copy(
                src_ref=send_buf, dst_ref=recv_buf,
                send_sem=rs_send_sems.at[j, s],
                recv_sem=rs_recv_sems.at[j, s],
                device_id=(right,), device_id_type=pl.DeviceIdType.MESH)
            rdma.start()
            rdma.wait()
            c = lax.rem(r - 2 - s + 2 * N_DEV, N_DEV)
            send_buf[...] = partial(c) + recv_buf[...]

            @pl.when(jnp.logical_not(
                jnp.logical_and(j == NB - 1, s == N_DEV - 2)))
            def _():
                pl.semaphore_signal(credit_sem, inc=1, device_id=(left,),
                                    device_id_type=pl.DeviceIdType.MESH)
            return carry

        lax.fori_loop(0, N_DEV - 1, rs_step, 0)

        co = pltpu.make_async_copy(
            send_buf, out_hbm.at[pl.ds(r * chunk + j * band, band), :],
            local_sem)
        co.start()
        co.wait()

        def ag_step(t, carry):
            a = lax.rem(r - t + N_DEV, N_DEV)
            src = out_hbm.at[pl.ds(a * chunk + j * band, band), :]
            rdma = pltpu.make_async_remote_copy(
                src_ref=src, dst_ref=src,
                send_sem=ag_send_sems.at[j, t],
                recv_sem=ag_recv_sems.at[j, t],
                device_id=(right,), device_id_type=pl.DeviceIdType.MESH)
            rdma.start()
            rdma.wait()
            return carry

        lax.fori_loop(0, N_DEV - 1, ag_step, 0)

    return pl.pallas_call(
        body,
        grid=(NB,),
        out_shape=jax.ShapeDtypeStruct((m, n), jnp.float32),
        in_specs=[pl.BlockSpec(memory_space=pl.ANY),
                  pl.BlockSpec(memory_space=pl.ANY)],
        out_specs=pl.BlockSpec(memory_space=pl.ANY),
        scratch_shapes=[
            pltpu.VMEM((k_shard, n), jnp.bfloat16),
            pltpu.VMEM((band, k_shard), jnp.bfloat16),
            pltpu.VMEM((band, n), jnp.float32),
            pltpu.VMEM((band, n), jnp.float32),
            pltpu.SemaphoreType.DMA,
            pltpu.SemaphoreType.DMA((NB, N_DEV - 1)),
            pltpu.SemaphoreType.DMA((NB, N_DEV - 1)),
            pltpu.SemaphoreType.DMA((NB, N_DEV - 1)),
            pltpu.SemaphoreType.DMA((NB, N_DEV - 1)),
            pltpu.SemaphoreType.REGULAR,
        ],
        compiler_params=pltpu.CompilerParams(
            collective_id=0, vmem_limit_bytes=63 * 1024 * 1024,
            dimension_semantics=("arbitrary",)),
    )(x, w_mat)


# device time: 1445597 ns/iter; 1.7323x vs baseline; 1.7323x over previous
import jax
import jax.numpy as jnp
from jax import lax
from jax.experimental import pallas as pl
from jax.experimental.pallas import tpu as pltpu

jax.config.update("jax_compilation_cache_dir", "/tmp/jaxcache")
jax.config.update("jax_persistent_cache_min_compile_time_secs", 1.0)

N_DEV = 4
NB = 2


def kernel(x, w_mat):
    x = x.astype(jnp.bfloat16)
    w_mat = w_mat.astype(jnp.bfloat16)
    m, k_shard = x.shape
    _, n = w_mat.shape
    chunk = m // N_DEV
    band = chunk // NB
    h = n // 2

    def body(x_hbm, w_hbm, out_hbm, w_vmem, x_vmem,
             send_r, recv_r, send_l, recv_l, local_sem,
             rsr_send, rsr_recv, rsl_send, rsl_recv,
             agr_send, agr_recv, agl_send, agl_recv,
             credit_r, credit_l):
        j = pl.program_id(0)
        r = lax.axis_index("i")
        right = lax.rem(r + 1, N_DEV)
        left = lax.rem(r + N_DEV - 1, N_DEV)

        @pl.when(j == 0)
        def _():
            bsem = pltpu.get_barrier_semaphore()
            pl.semaphore_signal(bsem, inc=1, device_id=(left,),
                                device_id_type=pl.DeviceIdType.MESH)
            pl.semaphore_signal(bsem, inc=1, device_id=(right,),
                                device_id_type=pl.DeviceIdType.MESH)
            pl.semaphore_wait(bsem, 2)
            cw = pltpu.make_async_copy(w_hbm, w_vmem, local_sem)
            cw.start()
            cw.wait()

        def load_x(c):
            cp = pltpu.make_async_copy(
                x_hbm.at[pl.ds(c * chunk + j * band, band), :], x_vmem,
                local_sem)
            cp.start()
            cp.wait()

        def partial_r(c):
            load_x(c)
            return jnp.dot(x_vmem[...], w_vmem[:, 0:h],
                           preferred_element_type=jnp.float32)

        def partial_l(c):
            load_x(c)
            return jnp.dot(x_vmem[...], w_vmem[:, h:n],
                           preferred_element_type=jnp.float32)

        send_r[...] = partial_r(lax.rem(r + N_DEV - 1, N_DEV))
        send_l[...] = partial_l(lax.rem(r + 1, N_DEV))

        def rs_step(s, carry):
            @pl.when(jnp.logical_or(j > 0, s > 0))
            def _():
                pl.semaphore_wait(credit_r, 1)
                pl.semaphore_wait(credit_l, 1)

            rdr = pltpu.make_async_remote_copy(
                src_ref=send_r, dst_ref=recv_r,
                send_sem=rsr_send.at[j, s], recv_sem=rsr_recv.at[j, s],
                device_id=(right,), device_id_type=pl.DeviceIdType.MESH)
            rdl = pltpu.make_async_remote_copy(
                src_ref=send_l, dst_ref=recv_l,
                send_sem=rsl_send.at[j, s], recv_sem=rsl_recv.at[j, s],
                device_id=(left,), device_id_type=pl.DeviceIdType.MESH)
            rdr.start()
            rdl.start()
            rdr.wait()
            rdl.wait()

            cr = lax.rem(r - 2 - s + 2 * N_DEV, N_DEV)
            send_r[...] = partial_r(cr) + recv_r[...]
            cl = lax.rem(r + 2 + s, N_DEV)
            send_l[...] = partial_l(cl) + recv_l[...]

            @pl.when(jnp.logical_not(
                jnp.logical_and(j == NB - 1, s == N_DEV - 2)))
            def _():
                pl.semaphore_signal(credit_r, inc=1, device_id=(left,),
                                    device_id_type=pl.DeviceIdType.MESH)
                pl.semaphore_signal(credit_l, inc=1, device_id=(right,),
                                    device_id_type=pl.DeviceIdType.MESH)
            return carry

        lax.fori_loop(0, N_DEV - 1, rs_step, 0)

        c1 = pltpu.make_async_copy(
            send_r, out_hbm.at[pl.ds(r * chunk + j * band, band), 0:h],
            local_sem)
        c1.start()
        c2 = pltpu.make_async_copy(
            send_l, out_hbm.at[pl.ds(r * chunk + j * band, band), h:n],
            local_sem)
        c2.start()
        c1.wait()
        c2.wait()

        def ag_step(t, carry):
            ar = lax.rem(r - t + N_DEV, N_DEV)
            srcr = out_hbm.at[pl.ds(ar * chunk + j * band, band), 0:h]
            rdr = pltpu.make_async_remote_copy(
                src_ref=srcr, dst_ref=srcr,
                send_sem=agr_send.at[j, t], recv_sem=agr_recv.at[j, t],
                device_id=(right,), device_id_type=pl.DeviceIdType.MESH)
            al = lax.rem(r + t, N_DEV)
            srcl = out_hbm.at[pl.ds(al * chunk + j * band, band), h:n]
            rdl = pltpu.make_async_remote_copy(
                src_ref=srcl, dst_ref=srcl,
                send_sem=agl_send.at[j, t], recv_sem=agl_recv.at[j, t],
                device_id=(left,), device_id_type=pl.DeviceIdType.MESH)
            rdr.start()
            rdl.start()
            rdr.wait()
            rdl.wait()
            return carry

        lax.fori_loop(0, N_DEV - 1, ag_step, 0)

    nsteps = (NB, N_DEV - 1)
    return pl.pallas_call(
        body,
        grid=(NB,),
        out_shape=jax.ShapeDtypeStruct((m, n), jnp.float32),
        in_specs=[pl.BlockSpec(memory_space=pl.ANY),
                  pl.BlockSpec(memory_space=pl.ANY)],
        out_specs=pl.BlockSpec(memory_space=pl.ANY),
        scratch_shapes=[
            pltpu.VMEM((k_shard, n), jnp.bfloat16),
            pltpu.VMEM((band, k_shard), jnp.bfloat16),
            pltpu.VMEM((band, h), jnp.float32),
            pltpu.VMEM((band, h), jnp.float32),
            pltpu.VMEM((band, h), jnp.float32),
            pltpu.VMEM((band, h), jnp.float32),
            pltpu.SemaphoreType.DMA,
            pltpu.SemaphoreType.DMA(nsteps),
            pltpu.SemaphoreType.DMA(nsteps),
            pltpu.SemaphoreType.DMA(nsteps),
            pltpu.SemaphoreType.DMA(nsteps),
            pltpu.SemaphoreType.DMA(nsteps),
            pltpu.SemaphoreType.DMA(nsteps),
            pltpu.SemaphoreType.DMA(nsteps),
            pltpu.SemaphoreType.DMA(nsteps),
            pltpu.SemaphoreType.REGULAR,
            pltpu.SemaphoreType.REGULAR,
        ],
        compiler_params=pltpu.CompilerParams(
            collective_id=0, vmem_limit_bytes=63 * 1024 * 1024,
            dimension_semantics=("arbitrary",)),
    )(x, w_mat)
